# Initial kernel scaffold; baseline (speedup 1.0000x reference)
import jax
import jax.numpy as jnp
from jax.experimental import pallas as pl


def kernel(v_idxs, h_idxs, virus, human, vb, hb, W, b):
    raise NotImplementedError("write your pallas kernel here")


# speedup vs baseline: 1.4759x; 1.4759x over previous
import functools

import jax
import jax.numpy as jnp
from jax import lax
from jax.experimental import pallas as pl
from jax.experimental.pallas import tpu as pltpu
from jax.experimental.pallas import tpu_sc as plsc

NC = 2
NS = 16
L = 16
NW = NC * NS

B = 16384
D = 128
CHUNK = 128
ROWS_PER_W = B // NW
NCHUNK = ROWS_PER_W // CHUNK
NSUB = D // L
NGROUP = CHUNK // L


def _gmf_body(vidx_hbm, hidx_hbm, virus_hbm, human_hbm, vb_hbm, hb_hbm,
              w_hbm, bsplat_hbm, out_hbm,
              vidx_v, hidx_v, vrows_v, hrows_v, vb_v, hb_v, w_v, b_v,
              out_v, sem):
    wid = lax.axis_index("s") * NC + lax.axis_index("c")

    pltpu.sync_copy(vidx_hbm.at[wid], vidx_v)
    pltpu.sync_copy(hidx_hbm.at[wid], hidx_v)
    pltpu.sync_copy(w_hbm, w_v)
    pltpu.sync_copy(bsplat_hbm, b_v)

    lane_iota = lax.iota(jnp.int32, L)

    def hsplat(x):
        for k in (1, 2, 4, 8):
            idx = lax.bitwise_xor(lane_iota, k)
            x = x + x.at[idx].get(mode="promise_in_bounds")
        return x

    wtot = w_v[pl.ds(0, L)]
    for c in range(1, NSUB):
        wtot = wtot + w_v[pl.ds(c * L, L)]
    wsplat = hsplat(wtot)
    bvec = b_v[...]

    def chunk_body(j, carry):
        wsplat, bvec = carry
        cp0 = pltpu.async_copy(virus_hbm.at[vidx_v.at[j]], vrows_v, sem)
        cp1 = pltpu.async_copy(human_hbm.at[hidx_v.at[j]], hrows_v, sem)
        cp2 = pltpu.async_copy(vb_hbm.at[vidx_v.at[j]], vb_v, sem)
        cp3 = pltpu.async_copy(hb_hbm.at[hidx_v.at[j]], hb_v, sem)
        cp0.wait()
        cp1.wait()
        cp2.wait()
        cp3.wait()

        def group_body(g, carry2):
            wsplat, bvec = carry2
            rbase = g * L
            wvecs = [w_v[pl.ds(c * L, L)] for c in range(NSUB)]
            dot = jnp.zeros((L,), jnp.float32)
            for r in range(L):
                row = rbase + r
                acc = (vrows_v[row, pl.ds(0, L)] * hrows_v[row, pl.ds(0, L)]
                       * wvecs[0])
                for c in range(1, NSUB):
                    acc = acc + (vrows_v[row, pl.ds(c * L, L)]
                                 * hrows_v[row, pl.ds(c * L, L)] * wvecs[c])
                dot = jnp.where(lane_iota == r, hsplat(acc), dot)
            vbg = vb_v[pl.ds(rbase, L)]
            hbg = hb_v[pl.ds(rbase, L)]
            z = dot + (vbg + hbg) * wsplat + bvec
            res = 1.0 / (1.0 + jnp.exp(-z))
            out_v[pl.ds(j * CHUNK + rbase, L)] = res
            return carry2

        return lax.fori_loop(0, NGROUP, group_body, carry)

    lax.fori_loop(0, NCHUNK, chunk_body, (wsplat, bvec))
    pltpu.sync_copy(out_v, out_hbm.at[pl.ds(wid * ROWS_PER_W, ROWS_PER_W)])


@jax.jit
def _gmf(vidx, hidx, virus, human, vb, hb, w, bsplat):
    mesh = plsc.VectorSubcoreMesh(core_axis_name="c", subcore_axis_name="s")
    run = functools.partial(
        pl.kernel,
        out_type=jax.ShapeDtypeStruct((B,), jnp.float32),
        mesh=mesh,
        scratch_types=[
            pltpu.VMEM((NCHUNK, CHUNK), jnp.int32),
            pltpu.VMEM((NCHUNK, CHUNK), jnp.int32),
            pltpu.VMEM((CHUNK, D), jnp.float32),
            pltpu.VMEM((CHUNK, D), jnp.float32),
            pltpu.VMEM((CHUNK,), jnp.float32),
            pltpu.VMEM((CHUNK,), jnp.float32),
            pltpu.VMEM((D,), jnp.float32),
            pltpu.VMEM((L,), jnp.float32),
            pltpu.VMEM((ROWS_PER_W,), jnp.float32),
            pltpu.SemaphoreType.DMA,
        ],
    )(_gmf_body)
    return run(vidx, hidx, virus, human, vb, hb, w, bsplat)


def kernel(v_idxs, h_idxs, virus, human, vb, hb, W, b):
    vidx = v_idxs.astype(jnp.int32).reshape(NW, NCHUNK, CHUNK)
    hidx = h_idxs.astype(jnp.int32).reshape(NW, NCHUNK, CHUNK)
    out = _gmf(vidx, hidx, virus, human,
               vb.reshape(-1), hb.reshape(-1), W.reshape(-1),
               jnp.broadcast_to(b.reshape(()), (L,)))
    return out.reshape(B, 1)


# speedup vs baseline: 1.7184x; 1.1643x over previous
import functools

import jax
import jax.numpy as jnp
from jax import lax
from jax.experimental import pallas as pl
from jax.experimental.pallas import tpu as pltpu
from jax.experimental.pallas import tpu_sc as plsc

NC = 2
NS = 16
L = 16
NW = NC * NS

B = 16384
D = 128
CHUNK = 128
ROWS_PER_W = B // NW
NCHUNK = ROWS_PER_W // CHUNK
NSUB = D // L
NGROUP = CHUNK // L


def _gmf_body(vidx_hbm, hidx_hbm, virus_hbm, human_hbm, vb_hbm, hb_hbm,
              w_hbm, bsplat_hbm, out_hbm,
              vidx_v, hidx_v, vrows0_v, hrows0_v, vb0_v, hb0_v,
              vrows1_v, hrows1_v, vb1_v, hb1_v, w_v, b_v,
              out_v, sem0, sem1):
    wid = lax.axis_index("s") * NC + lax.axis_index("c")

    pltpu.sync_copy(vidx_hbm.at[wid], vidx_v)
    pltpu.sync_copy(hidx_hbm.at[wid], hidx_v)
    pltpu.sync_copy(w_hbm, w_v)
    pltpu.sync_copy(bsplat_hbm, b_v)

    lane_iota = lax.iota(jnp.int32, L)

    def hsplat(x):
        for k in (1, 2, 4, 8):
            idx = lax.bitwise_xor(lane_iota, k)
            x = x + x.at[idx].get(mode="promise_in_bounds")
        return x

    wtot = w_v[pl.ds(0, L)]
    for c in range(1, NSUB):
        wtot = wtot + w_v[pl.ds(c * L, L)]
    wsplat = hsplat(wtot)
    bvec = b_v[...]

    bufs = ((vrows0_v, hrows0_v, vb0_v, hb0_v, sem0),
            (vrows1_v, hrows1_v, vb1_v, hb1_v, sem1))

    def issue(j, buf):
        vrows_v, hrows_v, vb_v, hb_v, sem = buf
        return (pltpu.async_copy(virus_hbm.at[vidx_v.at[j]], vrows_v, sem),
                pltpu.async_copy(human_hbm.at[hidx_v.at[j]], hrows_v, sem),
                pltpu.async_copy(vb_hbm.at[vidx_v.at[j]], vb_v, sem),
                pltpu.async_copy(hb_hbm.at[hidx_v.at[j]], hb_v, sem))

    def compute(j, buf):
        vrows_v, hrows_v, vb_v, hb_v, _ = buf

        def group_body(g, carry2):
            wsplat, bvec = carry2
            rbase = g * L
            wvecs = [w_v[pl.ds(c * L, L)] for c in range(NSUB)]
            dot = jnp.zeros((L,), jnp.float32)
            for r in range(L):
                row = rbase + r
                acc = (vrows_v[row, pl.ds(0, L)] * hrows_v[row, pl.ds(0, L)]
                       * wvecs[0])
                for c in range(1, NSUB):
                    acc = acc + (vrows_v[row, pl.ds(c * L, L)]
                                 * hrows_v[row, pl.ds(c * L, L)] * wvecs[c])
                dot = jnp.where(lane_iota == r, hsplat(acc), dot)
            vbg = vb_v[pl.ds(rbase, L)]
            hbg = hb_v[pl.ds(rbase, L)]
            z = dot + (vbg + hbg) * wsplat + bvec
            res = 1.0 / (1.0 + jnp.exp(-z))
            out_v[pl.ds(j * CHUNK + rbase, L)] = res
            return carry2

        lax.fori_loop(0, NGROUP, group_body, (wsplat, bvec))

    pending = issue(0, bufs[0])
    for j in range(NCHUNK):
        nxt = issue(j + 1, bufs[(j + 1) % 2]) if j + 1 < NCHUNK else None
        for cp in pending:
            cp.wait()
        compute(j, bufs[j % 2])
        pending = nxt
    pltpu.sync_copy(out_v, out_hbm.at[pl.ds(wid * ROWS_PER_W, ROWS_PER_W)])


@jax.jit
def _gmf(vidx, hidx, virus, human, vb, hb, w, bsplat):
    mesh = plsc.VectorSubcoreMesh(core_axis_name="c", subcore_axis_name="s")
    run = functools.partial(
        pl.kernel,
        out_type=jax.ShapeDtypeStruct((B,), jnp.float32),
        mesh=mesh,
        scratch_types=[
            pltpu.VMEM((NCHUNK, CHUNK), jnp.int32),
            pltpu.VMEM((NCHUNK, CHUNK), jnp.int32),
            pltpu.VMEM((CHUNK, D), jnp.float32),
            pltpu.VMEM((CHUNK, D), jnp.float32),
            pltpu.VMEM((CHUNK,), jnp.float32),
            pltpu.VMEM((CHUNK,), jnp.float32),
            pltpu.VMEM((CHUNK, D), jnp.float32),
            pltpu.VMEM((CHUNK, D), jnp.float32),
            pltpu.VMEM((CHUNK,), jnp.float32),
            pltpu.VMEM((CHUNK,), jnp.float32),
            pltpu.VMEM((D,), jnp.float32),
            pltpu.VMEM((L,), jnp.float32),
            pltpu.VMEM((ROWS_PER_W,), jnp.float32),
            pltpu.SemaphoreType.DMA,
            pltpu.SemaphoreType.DMA,
        ],
    )(_gmf_body)
    return run(vidx, hidx, virus, human, vb, hb, w, bsplat)


def kernel(v_idxs, h_idxs, virus, human, vb, hb, W, b):
    vidx = v_idxs.astype(jnp.int32).reshape(NW, NCHUNK, CHUNK)
    hidx = h_idxs.astype(jnp.int32).reshape(NW, NCHUNK, CHUNK)
    out = _gmf(vidx, hidx, virus, human,
               vb.reshape(-1), hb.reshape(-1), W.reshape(-1),
               jnp.broadcast_to(b.reshape(()), (L,)))
    return out.reshape(B, 1)


# speedup vs baseline: 1.9317x; 1.1241x over previous
import functools

import jax
import jax.numpy as jnp
from jax import lax
from jax.experimental import pallas as pl
from jax.experimental.pallas import tpu as pltpu
from jax.experimental.pallas import tpu_sc as plsc

NC = 2
NS = 16
L = 16
NW = NC * NS

B = 16384
D = 128
CHUNK = 128
ROWS_PER_W = B // NW
NCHUNK = ROWS_PER_W // CHUNK
NSUB = D // L
NGROUP = CHUNK // L


def _gmf_body(vidx_hbm, hidx_hbm, virus_hbm, human_hbm, vb_hbm, hb_hbm,
              w_hbm, b_hbm, out_hbm,
              vidx_v, hidx_v, vrows_v, hrows_v, vb_v, hb_v, w_v, b_v,
              out_v, sem):
    wid = lax.axis_index("s") * NC + lax.axis_index("c")

    pltpu.sync_copy(vidx_hbm.at[wid], vidx_v)
    pltpu.sync_copy(hidx_hbm.at[wid], hidx_v)
    pltpu.sync_copy(w_hbm, w_v)
    pltpu.sync_copy(b_hbm, b_v.at[pl.ds(0, 1)])

    lane_iota = lax.iota(jnp.int32, L)

    def hsplat(x):
        for k in (1, 2, 4, 8):
            idx = lax.bitwise_xor(lane_iota, k)
            x = x + x.at[idx].get(mode="promise_in_bounds")
        return x

    wtot = w_v[pl.ds(0, L)]
    for c in range(1, NSUB):
        wtot = wtot + w_v[pl.ds(c * L, L)]
    wsplat = hsplat(wtot)
    bvec = hsplat(jnp.where(lane_iota == 0, b_v[...], 0.0))

    def issue(j, s):
        pltpu.async_copy(virus_hbm.at[vidx_v.at[j, 0]], vrows_v.at[s],
                         sem.at[s])
        pltpu.async_copy(human_hbm.at[hidx_v.at[j, 0]], hrows_v.at[s],
                         sem.at[s])
        pltpu.async_copy(vb_hbm.at[vidx_v.at[j]], vb_v.at[s], sem.at[s])
        pltpu.async_copy(hb_hbm.at[hidx_v.at[j]], hb_v.at[s], sem.at[s])

    def drain(j, s):
        pltpu.make_async_copy(virus_hbm.at[vidx_v.at[j, 0]], vrows_v.at[s],
                              sem.at[s]).wait()
        pltpu.make_async_copy(human_hbm.at[hidx_v.at[j, 0]], hrows_v.at[s],
                              sem.at[s]).wait()
        pltpu.make_async_copy(vb_hbm.at[vidx_v.at[j]], vb_v.at[s],
                              sem.at[s]).wait()
        pltpu.make_async_copy(hb_hbm.at[hidx_v.at[j]], hb_v.at[s],
                              sem.at[s]).wait()

    issue(0, 0)

    def chunk_body(j, carry):
        wsplat, bvec = carry
        s = lax.rem(j, 2)
        ns = lax.rem(j + 1, 2)

        @pl.when(j < NCHUNK - 1)
        def _():
            issue(j + 1, ns)

        drain(j, s)

        def group_body(g, carry2):
            wsplat, bvec = carry2
            rbase = g * L
            wvecs = [w_v[pl.ds(c * L, L)] for c in range(NSUB)]
            dot = jnp.zeros((L,), jnp.float32)
            for r in range(L):
                row = rbase + r
                acc = (vrows_v[s, row, pl.ds(0, L)]
                       * hrows_v[s, row, pl.ds(0, L)] * wvecs[0])
                for c in range(1, NSUB):
                    acc = acc + (vrows_v[s, row, pl.ds(c * L, L)]
                                 * hrows_v[s, row, pl.ds(c * L, L)]
                                 * wvecs[c])
                dot = jnp.where(lane_iota == r, hsplat(acc), dot)
            vbg = vb_v[s, 0, pl.ds(rbase, L)]
            hbg = hb_v[s, 0, pl.ds(rbase, L)]
            z = dot + (vbg + hbg) * wsplat + bvec
            res = 1.0 / (1.0 + jnp.exp(-z))
            out_v[pl.ds(j * CHUNK + rbase, L)] = res
            return carry2

        return lax.fori_loop(0, NGROUP, group_body, (wsplat, bvec))

    lax.fori_loop(0, NCHUNK, chunk_body, (wsplat, bvec))
    pltpu.sync_copy(out_v, out_hbm.at[pl.ds(wid * ROWS_PER_W, ROWS_PER_W)])


@jax.jit
def _gmf(vidx, hidx, virus, human, vb, hb, w, b):
    mesh = plsc.VectorSubcoreMesh(core_axis_name="c", subcore_axis_name="s")
    run = functools.partial(
        pl.kernel,
        out_type=jax.ShapeDtypeStruct((B,), jnp.float32),
        mesh=mesh,
        scratch_types=[
            pltpu.VMEM((NCHUNK, 1, CHUNK), jnp.int32),
            pltpu.VMEM((NCHUNK, 1, CHUNK), jnp.int32),
            pltpu.VMEM((2, CHUNK, D), jnp.float32),
            pltpu.VMEM((2, CHUNK, D), jnp.float32),
            pltpu.VMEM((2, 1, CHUNK), jnp.float32),
            pltpu.VMEM((2, 1, CHUNK), jnp.float32),
            pltpu.VMEM((D,), jnp.float32),
            pltpu.VMEM((L,), jnp.float32),
            pltpu.VMEM((ROWS_PER_W,), jnp.float32),
            pltpu.SemaphoreType.DMA((2,)),
        ],
    )(_gmf_body)
    return run(vidx, hidx, virus, human, vb, hb, w, b)


def kernel(v_idxs, h_idxs, virus, human, vb, hb, W, b):
    vidx = v_idxs.astype(jnp.int32).reshape(NW, NCHUNK, 1, CHUNK)
    hidx = h_idxs.astype(jnp.int32).reshape(NW, NCHUNK, 1, CHUNK)
    out = _gmf(vidx, hidx, virus, human, vb.reshape(1, -1), hb.reshape(1, -1),
               W.reshape(-1), b)
    return out.reshape(B, 1)


# speedup vs baseline: 1.9635x; 1.0165x over previous
import functools

import jax
import jax.numpy as jnp
from jax import lax
from jax.experimental import pallas as pl
from jax.experimental.pallas import tpu as pltpu
from jax.experimental.pallas import tpu_sc as plsc

NC = 2
NS = 16
L = 16
NW = NC * NS

B = 16384
D = 128
CHUNK = 128
ROWS_PER_W = B // NW
NCHUNK = ROWS_PER_W // CHUNK
NSUB = D // L
NGROUP = CHUNK // L


def _gmf_body(vidx_hbm, hidx_hbm, virus_hbm, human_hbm, vb_hbm, hb_hbm,
              w_hbm, b_hbm, out_hbm,
              vidx_v, hidx_v, vrows_v, hrows_v, vb_v, hb_v, w_v, b_v,
              ws_v, pscr_v, out_v, sem):
    wid = lax.axis_index("s") * NC + lax.axis_index("c")

    pltpu.sync_copy(vidx_hbm.at[wid], vidx_v)
    pltpu.sync_copy(hidx_hbm.at[wid], hidx_v)
    pltpu.sync_copy(w_hbm, w_v)
    pltpu.sync_copy(b_hbm, b_v.at[pl.ds(0, 1)])

    lane_iota = lax.iota(jnp.int32, L)

    def hsplat(x):
        for k in (1, 2, 4, 8):
            idx = lax.bitwise_xor(lane_iota, k)
            x = x + x.at[idx].get(mode="promise_in_bounds")
        return x

    wtot = w_v[pl.ds(0, L)]
    for c in range(1, NSUB):
        wtot = wtot + w_v[pl.ds(c * L, L)]
    bvec0 = hsplat(jnp.where(lane_iota == 0, b_v[...], 0.0))
    ws_v[...] = hsplat(wtot)
    b_v[...] = bvec0

    merge_masks = [(lane_iota & h) == 0 for h in (8, 4, 2, 1)]
    merge_perms = [lax.bitwise_xor(lane_iota, h) for h in (8, 4, 2, 1)]
    bitrev = [0, 8, 4, 12, 2, 10, 6, 14, 1, 9, 5, 13, 3, 11, 7, 15]

    def issue(j, s):
        pltpu.async_copy(virus_hbm.at[vidx_v.at[j, 0]], vrows_v.at[s],
                         sem.at[s])
        pltpu.async_copy(human_hbm.at[hidx_v.at[j, 0]], hrows_v.at[s],
                         sem.at[s])
        pltpu.async_copy(vb_hbm.at[vidx_v.at[j]], vb_v.at[s], sem.at[s])
        pltpu.async_copy(hb_hbm.at[hidx_v.at[j]], hb_v.at[s], sem.at[s])

    def drain(j, s):
        pltpu.make_async_copy(virus_hbm.at[vidx_v.at[j, 0]], vrows_v.at[s],
                              sem.at[s]).wait()
        pltpu.make_async_copy(human_hbm.at[hidx_v.at[j, 0]], hrows_v.at[s],
                              sem.at[s]).wait()
        pltpu.make_async_copy(vb_hbm.at[vidx_v.at[j]], vb_v.at[s],
                              sem.at[s]).wait()
        pltpu.make_async_copy(hb_hbm.at[hidx_v.at[j]], hb_v.at[s],
                              sem.at[s]).wait()

    issue(0, 0)

    def chunk_body(j, carry):
        s = lax.rem(j, 2)
        ns = lax.rem(j + 1, 2)

        @pl.when(j < NCHUNK - 1)
        def _():
            issue(j + 1, ns)

        drain(j, s)

        def group_body(g, carry2):
            rbase = g * L
            wvecs = [w_v[pl.ds(c * L, L)] for c in range(NSUB)]

            def row_body(rr, carry3):
                row = rbase + rr
                acc = (vrows_v[s, row, pl.ds(0, L)]
                       * hrows_v[s, row, pl.ds(0, L)] * wvecs[0])
                for c in range(1, NSUB):
                    acc = acc + (vrows_v[s, row, pl.ds(c * L, L)]
                                 * hrows_v[s, row, pl.ds(c * L, L)]
                                 * wvecs[c])
                pscr_v[rr, :] = acc
                return carry3

            lax.fori_loop(0, L, row_body, 0, unroll=4)

            def tree(lo, span):
                if span == 1:
                    return pscr_v[bitrev[lo], :]
                half = span // 2
                x = tree(lo, half)
                y = tree(lo + half, half)
                stage = {8: 0, 4: 1, 2: 2, 1: 3}[L // span]
                prm = merge_perms[stage]
                u = x + x.at[prm].get(mode="promise_in_bounds")
                v = y + y.at[prm].get(mode="promise_in_bounds")
                return jnp.where(merge_masks[stage], u, v)

            dot = tree(0, L)
            vbg = vb_v[s, 0, pl.ds(rbase, L)]
            hbg = hb_v[s, 0, pl.ds(rbase, L)]
            z = dot + (vbg + hbg) * ws_v[...] + b_v[...]
            res = 1.0 / (1.0 + jnp.exp(-z))
            out_v[pl.ds(j * CHUNK + rbase, L)] = res
            return carry2

        return lax.fori_loop(0, NGROUP, group_body, 0)

    lax.fori_loop(0, NCHUNK, chunk_body, 0)
    pltpu.sync_copy(out_v, out_hbm.at[pl.ds(wid * ROWS_PER_W, ROWS_PER_W)])


@jax.jit
def _gmf(vidx, hidx, virus, human, vb, hb, w, b):
    mesh = plsc.VectorSubcoreMesh(core_axis_name="c", subcore_axis_name="s")
    run = functools.partial(
        pl.kernel,
        out_type=jax.ShapeDtypeStruct((B,), jnp.float32),
        mesh=mesh,
        scratch_types=[
            pltpu.VMEM((NCHUNK, 1, CHUNK), jnp.int32),
            pltpu.VMEM((NCHUNK, 1, CHUNK), jnp.int32),
            pltpu.VMEM((2, CHUNK, D), jnp.float32),
            pltpu.VMEM((2, CHUNK, D), jnp.float32),
            pltpu.VMEM((2, 1, CHUNK), jnp.float32),
            pltpu.VMEM((2, 1, CHUNK), jnp.float32),
            pltpu.VMEM((D,), jnp.float32),
            pltpu.VMEM((L,), jnp.float32),
            pltpu.VMEM((L,), jnp.float32),
            pltpu.VMEM((L, L), jnp.float32),
            pltpu.VMEM((ROWS_PER_W,), jnp.float32),
            pltpu.SemaphoreType.DMA((2,)),
        ],
    )(_gmf_body)
    return run(vidx, hidx, virus, human, vb, hb, w, b)


def kernel(v_idxs, h_idxs, virus, human, vb, hb, W, b):
    vidx = v_idxs.astype(jnp.int32).reshape(NW, NCHUNK, 1, CHUNK)
    hidx = h_idxs.astype(jnp.int32).reshape(NW, NCHUNK, 1, CHUNK)
    out = _gmf(vidx, hidx, virus, human, vb.reshape(1, -1), hb.reshape(1, -1),
               W.reshape(-1), b)
    return out.reshape(B, 1)
